# Initial kernel scaffold; baseline (speedup 1.0000x reference)
import jax
import jax.numpy as jnp
from jax.experimental import pallas as pl


def kernel(x, table):
    raise NotImplementedError("write your pallas kernel here")


# speedup vs baseline: 4.5430x; 4.5430x over previous
import functools

import jax
import jax.numpy as jnp
from jax import lax
from jax.experimental import pallas as pl
from jax.experimental.pallas import tpu as pltpu
from jax.experimental.pallas import tpu_sc as plsc

_D = 64
_BATCH = 4096
_HIST = 50
_B = _BATCH * _HIST
_NW = 32
_PER_W = _B // _NW
_C = 128
_G = _PER_W // _C

_mesh = plsc.VectorSubcoreMesh(core_axis_name="c", subcore_axis_name="s")


@functools.partial(
    pl.kernel,
    mesh=_mesh,
    out_type=jax.ShapeDtypeStruct((_B, _D), jnp.float32),
    scratch_types=[
        pltpu.VMEM((_G, _C), jnp.int32),
        pltpu.VMEM((2, _C, _D), jnp.float32),
        pltpu.SemaphoreType.DMA,
        pltpu.SemaphoreType.DMA,
    ],
    compiler_params=pltpu.CompilerParams(use_tc_tiling_on_sc=False),
)
def _embed(table_hbm, x_hbm, out_hbm, idx_v, rows_v, sem0, sem1):
    wid = lax.axis_index("s") * 2 + lax.axis_index("c")
    base = wid * _PER_W
    sems = (sem0, sem1)

    pltpu.sync_copy(x_hbm.at[wid], idx_v)

    pltpu.async_copy(table_hbm.at[idx_v.at[0]], rows_v.at[0], sems[0])

    def outer(t, carry):
        for b in range(2):
            g = 2 * t + b
            nb = (b + 1) % 2

            @pl.when(g + 1 < _G)
            def _():
                pltpu.async_copy(
                    table_hbm.at[idx_v.at[g + 1]], rows_v.at[nb], sems[nb]
                )

            pltpu.make_async_copy(
                table_hbm.at[idx_v.at[0]], rows_v.at[b], sems[b]
            ).wait()
            pltpu.sync_copy(rows_v.at[b], out_hbm.at[pl.ds(base + g * _C, _C)])
        return carry

    lax.fori_loop(0, _G // 2, outer, 0)


def kernel(x, table):
    xw = x.reshape(_NW, _G, _C).astype(jnp.int32)
    out = _embed(table, xw)
    return out.reshape(_BATCH, _HIST, _D)


# speedup vs baseline: 4.6764x; 1.0294x over previous
import functools

import jax
import jax.numpy as jnp
from jax import lax
from jax.experimental import pallas as pl
from jax.experimental.pallas import tpu as pltpu
from jax.experimental.pallas import tpu_sc as plsc

_D = 64
_BATCH = 4096
_HIST = 50
_B = _BATCH * _HIST
_NW = 32
_PER_W = _B // _NW
_C = 128
_G = _PER_W // _C
_NBUF = 5

_mesh = plsc.VectorSubcoreMesh(core_axis_name="c", subcore_axis_name="s")


@functools.partial(
    pl.kernel,
    mesh=_mesh,
    out_type=jax.ShapeDtypeStruct((_B, _D), jnp.float32),
    scratch_types=[
        pltpu.VMEM((_G, _C), jnp.int32),
        pltpu.VMEM((_NBUF, _C, _D), jnp.float32),
        pltpu.SemaphoreType.DMA((_NBUF,)),
        pltpu.SemaphoreType.DMA((_NBUF,)),
    ],
    compiler_params=pltpu.CompilerParams(use_tc_tiling_on_sc=False),
)
def _embed(table_hbm, x_hbm, out_hbm, idx_v, rows_v, gsem, wsem):
    wid = lax.axis_index("s") * 2 + lax.axis_index("c")
    base = wid * _PER_W

    pltpu.sync_copy(x_hbm.at[wid], idx_v)

    for b in range(_NBUF - 1):
        pltpu.async_copy(table_hbm.at[idx_v.at[b]], rows_v.at[b], gsem.at[b])

    def outer(t, carry):
        for b in range(_NBUF):
            g = _NBUF * t + b
            nb = (b + _NBUF - 1) % _NBUF

            pltpu.make_async_copy(
                table_hbm.at[idx_v.at[0]], rows_v.at[b], gsem.at[b]
            ).wait()
            pltpu.async_copy(
                rows_v.at[b], out_hbm.at[pl.ds(base + g * _C, _C)], wsem.at[b]
            )

            @pl.when(g + _NBUF - 1 < _G)
            def _():
                @pl.when(g >= 1)
                def _():
                    pltpu.make_async_copy(
                        rows_v.at[nb], out_hbm.at[pl.ds(base, _C)], wsem.at[nb]
                    ).wait()

                pltpu.async_copy(
                    table_hbm.at[idx_v.at[g + _NBUF - 1]],
                    rows_v.at[nb],
                    gsem.at[nb],
                )
        return carry

    lax.fori_loop(0, _G // _NBUF, outer, 0)

    for b in range(_NBUF):
        pltpu.make_async_copy(
            rows_v.at[b], out_hbm.at[pl.ds(base, _C)], wsem.at[b]
        ).wait()


def kernel(x, table):
    xw = x.reshape(_NW, _G, _C).astype(jnp.int32)
    out = _embed(table, xw)
    return out.reshape(_BATCH, _HIST, _D)


# speedup vs baseline: 4.6852x; 1.0019x over previous
import functools

import jax
import jax.numpy as jnp
from jax import lax
from jax.experimental import pallas as pl
from jax.experimental.pallas import tpu as pltpu
from jax.experimental.pallas import tpu_sc as plsc

_D = 64
_BATCH = 4096
_HIST = 50
_B = _BATCH * _HIST
_NW = 32
_PER_W = _B // _NW
_C = 256
_G = _PER_W // _C
_NBUF = 5

_mesh = plsc.VectorSubcoreMesh(core_axis_name="c", subcore_axis_name="s")


@functools.partial(
    pl.kernel,
    mesh=_mesh,
    out_type=jax.ShapeDtypeStruct((_B, _D), jnp.float32),
    scratch_types=[
        pltpu.VMEM((_PER_W,), jnp.int32),
        pltpu.VMEM((_NBUF, _C, _D), jnp.float32),
        pltpu.SemaphoreType.DMA((_NBUF,)),
        pltpu.SemaphoreType.DMA((_NBUF,)),
    ],
    compiler_params=pltpu.CompilerParams(use_tc_tiling_on_sc=False),
)
def _embed(table_hbm, x_hbm, out_hbm, idx_v, rows_v, gsem, wsem):
    wid = lax.axis_index("s") * 2 + lax.axis_index("c")
    base = wid * _PER_W

    pltpu.sync_copy(x_hbm.at[wid], idx_v)

    def fire(g, b):
        pltpu.async_copy(
            table_hbm.at[idx_v.at[pl.ds(g * _C, _C)]], rows_v.at[b], gsem.at[b]
        )

    for b in range(_NBUF - 1):
        fire(b, b)

    def outer(t, carry):
        for b in range(_NBUF):
            g = _NBUF * t + b
            nb = (b + _NBUF - 1) % _NBUF

            pltpu.make_async_copy(
                table_hbm.at[idx_v.at[pl.ds(0, _C)]], rows_v.at[b], gsem.at[b]
            ).wait()
            pltpu.async_copy(
                rows_v.at[b], out_hbm.at[pl.ds(base + g * _C, _C)], wsem.at[b]
            )

            @pl.when(g + _NBUF - 1 < _G)
            def _():
                @pl.when(g >= 1)
                def _():
                    pltpu.make_async_copy(
                        rows_v.at[nb], out_hbm.at[pl.ds(base, _C)], wsem.at[nb]
                    ).wait()

                fire(g + _NBUF - 1, nb)
        return carry

    lax.fori_loop(0, _G // _NBUF, outer, 0)

    for b in range(_NBUF):
        pltpu.make_async_copy(
            rows_v.at[b], out_hbm.at[pl.ds(base, _C)], wsem.at[b]
        ).wait()


def kernel(x, table):
    xw = x.reshape(_NW, _PER_W).astype(jnp.int32)
    out = _embed(table, xw)
    return out.reshape(_BATCH, _HIST, _D)
